# baseline (speedup 1.0000x reference)
import functools

import jax
import jax.numpy as jnp
from jax import lax
from jax.experimental import pallas as pl
from jax.experimental.pallas import tpu as pltpu
from jax.experimental.pallas import tpu_sc as plsc

_BUCKETS = 1000000
_BIGRAM_DIM = 32
_MODEL_DIM = 1024

_NC = 2
_NS = 16
_NW = _NC * _NS
_LANES = 16


def _sc_hash_gather(tok_flat, shf_flat, embed_W):
    n = tok_flat.shape[0]
    bpw = n // _NW
    n_vec = bpw // _LANES
    n_gather = bpw // 128
    mesh = plsc.VectorSubcoreMesh(core_axis_name="c", subcore_axis_name="s")

    @functools.partial(
        pl.kernel,
        mesh=mesh,
        out_type=jax.ShapeDtypeStruct((n, _BIGRAM_DIM), jnp.float32),
        scratch_types=[
            pltpu.VMEM((bpw,), jnp.int32),
            pltpu.VMEM((bpw,), jnp.int32),
            pltpu.VMEM((n_gather, 128), jnp.int32),
            pltpu.VMEM((bpw, _BIGRAM_DIM), jnp.float32),
            pltpu.SemaphoreType.DMA,
        ],
        compiler_params=pltpu.CompilerParams(use_tc_tiling_on_sc=False),
    )
    def k(t_hbm, s_hbm, table_hbm, out_hbm, tok_v, shf_v, idx_v, rows_v, sem):
        wid = lax.axis_index("s") * _NC + lax.axis_index("c")
        base = wid * bpw
        pltpu.sync_copy(t_hbm.at[pl.ds(base, bpw)], tok_v)
        pltpu.sync_copy(s_hbm.at[pl.ds(base, bpw)], shf_v)
        mod = jnp.int32(_BUCKETS - 1)
        for i in range(n_vec):
            t = tok_v[pl.ds(i * _LANES, _LANES)]
            s = shf_v[pl.ds(i * _LANES, _LANES)]
            h = jnp.int32(36313) * t + jnp.int32(27191) * s
            r = lax.rem(h, mod)
            r = jnp.where(r < 0, r + mod, r)
            idx_v[i // 8, pl.ds((i % 8) * _LANES, _LANES)] = r
        copies = [
            pltpu.make_async_copy(
                table_hbm.at[idx_v.at[j]],
                rows_v.at[pl.ds(j * 128, 128)],
                sem,
            )
            for j in range(n_gather)
        ]
        for c in copies:
            c.start()
        for c in copies:
            c.wait()
        pltpu.sync_copy(rows_v, out_hbm.at[pl.ds(base, bpw)])

    return k(tok_flat, shf_flat, embed_W)


def _tc_project(gathered, proj_Wt, scale, block_m=1024):
    n = gathered.shape[0]
    d = proj_Wt.shape[1]

    def body(s_ref, a_ref, p_ref, o_ref):
        o_ref[...] = (
            jnp.dot(a_ref[...], p_ref[...], preferred_element_type=jnp.float32)
            * s_ref[0]
        )

    return pl.pallas_call(
        body,
        grid=(n // block_m,),
        in_specs=[
            pl.BlockSpec(memory_space=pltpu.SMEM),
            pl.BlockSpec((block_m, _BIGRAM_DIM), lambda i: (i, 0)),
            pl.BlockSpec((_BIGRAM_DIM, d), lambda i: (0, 0)),
        ],
        out_specs=pl.BlockSpec((block_m, d), lambda i: (i, 0)),
        out_shape=jax.ShapeDtypeStruct((n, d), jnp.float32),
        compiler_params=pltpu.CompilerParams(
            dimension_semantics=("parallel",),
        ),
    )(jnp.reshape(scale, (1,)), gathered, proj_Wt)


def kernel(token_ids, embed_W, proj_W, scale):
    b, s = token_ids.shape
    t = token_ids.astype(jnp.int32)
    mod = jnp.int32(_BUCKETS - 1)
    shifted = jnp.concatenate(
        [jnp.full((b, 1), mod, dtype=jnp.int32), t[:, :-1]], axis=1
    )
    gathered = _sc_hash_gather(t.reshape(-1), shifted.reshape(-1), embed_W)
    out = _tc_project(gathered, proj_W.T, scale)
    return out.reshape(b, s, _MODEL_DIM)
